# Initial kernel scaffold; baseline (speedup 1.0000x reference)
import jax
import jax.numpy as jnp
from jax.experimental import pallas as pl


def kernel(node_feats, batch_index, Q):
    raise NotImplementedError("write your pallas kernel here")


# speedup vs baseline: 27.3964x; 27.3964x over previous
import functools
import jax
import jax.numpy as jnp
import numpy as np
from jax.experimental import pallas as pl
from jax.experimental.pallas import tpu as pltpu


def _body(feats_ref, bidx_ref, q_ref, out_ref, m_ref, s_ref, acc_ref,
          *, blk, b, d):
    i = pl.program_id(0)
    n = pl.num_programs(0)

    @pl.when(i == 0)
    def _init():
        m_ref[...] = jnp.full_like(m_ref, -jnp.inf)
        s_ref[...] = jnp.zeros_like(s_ref)
        acc_ref[...] = jnp.zeros_like(acc_ref)

    feats = feats_ref[...]
    bidx = bidx_ref[0, 0, :]
    q = q_ref[...]
    scores = jax.lax.dot_general(
        feats, q, (((1,), (1,)), ((), ())),
        preferred_element_type=jnp.float32) * (1.0 / np.sqrt(d))
    onehot = bidx[:, None] == jax.lax.broadcasted_iota(jnp.int32, (1, b), 1)
    smask = jnp.where(onehot, scores, -jnp.inf)
    bmax = jnp.max(smask, axis=0)
    m_old = m_ref[0, :]
    m_new = jnp.maximum(m_old, bmax)
    scale = jnp.where(m_old == -jnp.inf, 0.0, jnp.exp(m_old - m_new))
    e = jnp.where(onehot, jnp.exp(scores - m_new[None, :]), 0.0)
    m_ref[0, :] = m_new
    s_ref[0, :] = s_ref[0, :] * scale + jnp.sum(e, axis=0)
    acc_ref[...] = acc_ref[...] * scale[:, None] + jax.lax.dot_general(
        e, feats, (((0,), (0,)), ((), ())),
        preferred_element_type=jnp.float32)

    @pl.when(i != n - 1)
    def _zeros():
        out_ref[...] = jnp.zeros((blk, d), jnp.float32)

    @pl.when(i == n - 1)
    def _final():
        s = s_ref[0, :]
        recip = jnp.where(s > 0.0, 1.0, 0.0) / jnp.where(s > 0.0, s, 1.0)
        h = acc_ref[...] * recip[:, None]
        out_ref[...] = jnp.pad(h, ((0, blk - b), (0, 0)))


@jax.jit
def kernel(node_feats, batch_index, Q):
    v, d = node_feats.shape
    b = Q.shape[0]
    blk = 2048
    n = v // blk
    bidx3 = batch_index.reshape(n, 1, blk)

    body = functools.partial(_body, blk=blk, b=b, d=d)
    out = pl.pallas_call(
        body,
        grid=(n,),
        in_specs=[
            pl.BlockSpec((blk, d), lambda i: (n - 1 - i, 0)),
            pl.BlockSpec((1, 1, blk), lambda i: (n - 1 - i, 0, 0)),
            pl.BlockSpec((b, d), lambda i: (0, 0)),
        ],
        out_specs=pl.BlockSpec((blk, d), lambda i: (n - 1 - i, 0)),
        out_shape=jax.ShapeDtypeStruct((v, d), jnp.float32),
        scratch_shapes=[
            pltpu.VMEM((1, b), jnp.float32),
            pltpu.VMEM((1, b), jnp.float32),
            pltpu.VMEM((b, d), jnp.float32),
        ],
    )(node_feats, bidx3, Q)
    return out
